# Initial kernel scaffold; baseline (speedup 1.0000x reference)
import jax
import jax.numpy as jnp
from jax.experimental import pallas as pl


def kernel(x, edge_index, edge_weight, h0, W, lamda, alpha, l):
    raise NotImplementedError("write your pallas kernel here")


# speedup vs baseline: 4.1262x; 4.1262x over previous
import functools

import jax
import jax.numpy as jnp
from jax import lax
from jax.experimental import pallas as pl
from jax.experimental.pallas import tpu as pltpu
from jax.experimental.pallas import tpu_sc as plsc

N_NODES = 10000
N_EDGES = 160000
D = 256
DH = D // 2

NC = 2
NS = 16
LANES = 16

E_PER_TILE = N_EDGES // NS
CHUNK = 200
N_CHUNKS = E_PER_TILE // CHUNK
WB_TILES = 10
WB_ROWS = N_NODES // WB_TILES


def _spmm_body(x2, src_h, dst_h, ew_h, zeros_h, out2,
               acc, src_v, dst_v, w_v, rows_v, sem):
  cid = lax.axis_index("c")
  tid = lax.axis_index("s")

  row0 = tid * WB_ROWS

  @pl.when(tid < WB_TILES)
  def _zero():
    pltpu.sync_copy(zeros_h, acc.at[pl.ds(row0, WB_ROWS)])

  plsc.subcore_barrier()

  def chunk_step(j, carry):
    off = tid * E_PER_TILE + j * CHUNK
    pltpu.sync_copy(src_h.at[pl.ds(off, CHUNK)], src_v)
    pltpu.sync_copy(dst_h.at[pl.ds(off, CHUNK)], dst_v)
    pltpu.sync_copy(ew_h.at[pl.ds(off, CHUNK)], w_v)
    pltpu.async_copy(x2.at[cid].at[src_v], rows_v, sem).wait()

    def _scale16(base, lanes):
      wgrp = w_v[pl.ds(base, LANES)]
      for i in lanes:
        wb = lax.gather(
            wgrp, jnp.full((LANES, 1), i, jnp.int32),
            lax.GatherDimensionNumbers(offset_dims=(),
                                       collapsed_slice_dims=(0,),
                                       start_index_map=(0,)),
            slice_sizes=(1,), mode=lax.GatherScatterMode.PROMISE_IN_BOUNDS)
        r = base + i
        for k in range(DH // LANES):
          sl = pl.ds(k * LANES, LANES)
          rows_v[r, sl] = rows_v[r, sl] * wb

    def scale_group(g, c2):
      _scale16(g * LANES, range(LANES))
      return c2

    lax.fori_loop(0, CHUNK // LANES, scale_group, 0, unroll=False)
    if CHUNK % LANES:
      _scale16(CHUNK - LANES, range(LANES - CHUNK % LANES, LANES))

    pltpu.sync_copy(rows_v, acc.at[dst_v], add=True)
    return carry

  lax.fori_loop(0, N_CHUNKS, chunk_step, 0, unroll=False)
  plsc.subcore_barrier()

  @pl.when(tid < WB_TILES)
  def _writeback():
    pltpu.sync_copy(acc.at[pl.ds(row0, WB_ROWS)],
                    out2.at[cid].at[pl.ds(row0, WB_ROWS)])


@jax.jit
def _spmm(x2, src, dst, ew, zeros):
  mesh = plsc.VectorSubcoreMesh(core_axis_name="c", subcore_axis_name="s")
  f = pl.kernel(
      _spmm_body,
      out_type=jax.ShapeDtypeStruct((NC, N_NODES, DH), jnp.float32),
      mesh=mesh,
      scratch_types=[
          pltpu.VMEM_SHARED((N_NODES, DH), jnp.float32),
          pltpu.VMEM((CHUNK,), jnp.int32),
          pltpu.VMEM((CHUNK,), jnp.int32),
          pltpu.VMEM((CHUNK,), jnp.float32),
          pltpu.VMEM((CHUNK, DH), jnp.float32),
          pltpu.SemaphoreType.DMA,
      ],
  )
  return f(x2, src, dst, ew, zeros)


def _dense_body(hi_ref, h0_ref, w_ref, s_ref, out_ref):
  a = s_ref[0]
  th = s_ref[1]
  s = (1.0 - a) * hi_ref[...] + a * h0_ref[...]
  out_ref[...] = th * jnp.dot(s, w_ref[...],
                              preferred_element_type=jnp.float32) \
      + (1.0 - th) * s


@jax.jit
def _dense(hi, h0, W, scal):
  BM = 1000
  return pl.pallas_call(
      _dense_body,
      grid=(N_NODES // BM,),
      in_specs=[
          pl.BlockSpec((BM, D), lambda i: (i, 0)),
          pl.BlockSpec((BM, D), lambda i: (i, 0)),
          pl.BlockSpec((D, D), lambda i: (0, 0)),
          pl.BlockSpec(memory_space=pltpu.SMEM),
      ],
      out_specs=pl.BlockSpec((BM, D), lambda i: (i, 0)),
      out_shape=jax.ShapeDtypeStruct((N_NODES, D), jnp.float32),
  )(hi, h0, W, scal)


def kernel(x, edge_index, edge_weight, h0, W, lamda, alpha, l):
  src = edge_index[1].astype(jnp.int32)
  dst = edge_index[0].astype(jnp.int32)
  ew = edge_weight.astype(jnp.float32)
  x2 = jnp.stack([x[:, :DH], x[:, DH:]])
  zeros = jnp.zeros((WB_ROWS, DH), jnp.float32)

  hi2 = _spmm(x2, src, dst, ew, zeros)
  hi = jnp.concatenate([hi2[0], hi2[1]], axis=1)

  theta = jnp.log(lamda / l + 1.0).astype(jnp.float32)
  scal = jnp.stack([alpha.astype(jnp.float32), theta])
  return _dense(hi, h0, W, scal)
